# baseline (speedup 1.0000x reference)
import jax
import jax.numpy as jnp
from jax.experimental import pallas as pl

_BM = 512


def _router_kernel(x_ref, wt_ref, logits_ref, onehot_ref, pmax_ref):
    xb = x_ref[...].astype(jnp.bfloat16)
    logits = jnp.dot(xb, wt_ref[...], preferred_element_type=jnp.float32)
    rmax = jnp.max(logits, axis=1, keepdims=True)
    ssum = jnp.sum(jnp.exp(logits - rmax), axis=1, keepdims=True)
    pmax_ref[...] = 1.0 / ssum
    e = logits.shape[1]
    iota = jax.lax.broadcasted_iota(jnp.int32, logits.shape, 1)
    idx = jnp.min(jnp.where(logits == rmax, iota, e), axis=1, keepdims=True)
    onehot_ref[...] = (iota == idx).astype(jnp.int32)
    logits_ref[...] = logits


def kernel(hidden_states, W):
    b, s, h = hidden_states.shape
    e = W.shape[0]
    m = b * s
    x = hidden_states.reshape(m, h)
    wt = W.T.astype(jnp.bfloat16)

    logits, onehot, pmax = pl.pallas_call(
        _router_kernel,
        grid=(m // _BM,),
        in_specs=[
            pl.BlockSpec((_BM, h), lambda i: (i, 0)),
            pl.BlockSpec((h, e), lambda i: (0, 0)),
        ],
        out_specs=[
            pl.BlockSpec((_BM, e), lambda i: (i, 0)),
            pl.BlockSpec((_BM, e), lambda i: (i, 0)),
            pl.BlockSpec((_BM, 1), lambda i: (i, 0)),
        ],
        out_shape=[
            jax.ShapeDtypeStruct((m, e), jnp.float32),
            jax.ShapeDtypeStruct((m, e), jnp.int32),
            jax.ShapeDtypeStruct((m, 1), jnp.float32),
        ],
    )(x, wt)

    return (
        onehot.reshape(b, s, e),
        pmax.reshape(b, s, 1),
        logits.reshape(b, s, e),
    )


# speedup vs baseline: 1.0210x; 1.0210x over previous
import jax
import jax.numpy as jnp
from jax.experimental import pallas as pl

_BM = 1024


def _router_kernel(x_ref, wt_ref, logits_ref, onehot_ref, pmax_ref):
    xb = x_ref[...].astype(jnp.bfloat16)
    logits = jnp.dot(xb, wt_ref[...], preferred_element_type=jnp.float32)
    rmax = jnp.max(logits, axis=1, keepdims=True)
    ssum = jnp.sum(jnp.exp(logits - rmax), axis=1, keepdims=True)
    pmax_ref[...] = 1.0 / ssum
    e = logits.shape[1]
    iota = jax.lax.broadcasted_iota(jnp.int32, logits.shape, 1)
    idx = jnp.min(jnp.where(logits == rmax, iota, e), axis=1, keepdims=True)
    onehot_ref[...] = (iota == idx).astype(jnp.int32)
    logits_ref[...] = logits


def kernel(hidden_states, W):
    b, s, h = hidden_states.shape
    e = W.shape[0]
    m = b * s
    x = hidden_states.reshape(m, h)
    wt = W.T.astype(jnp.bfloat16)

    logits, onehot, pmax = pl.pallas_call(
        _router_kernel,
        grid=(m // _BM,),
        in_specs=[
            pl.BlockSpec((_BM, h), lambda i: (i, 0)),
            pl.BlockSpec((h, e), lambda i: (0, 0)),
        ],
        out_specs=[
            pl.BlockSpec((_BM, e), lambda i: (i, 0)),
            pl.BlockSpec((_BM, e), lambda i: (i, 0)),
            pl.BlockSpec((_BM, 1), lambda i: (i, 0)),
        ],
        out_shape=[
            jax.ShapeDtypeStruct((m, e), jnp.float32),
            jax.ShapeDtypeStruct((m, e), jnp.int32),
            jax.ShapeDtypeStruct((m, 1), jnp.float32),
        ],
    )(x, wt)

    return (
        onehot.reshape(b, s, e),
        pmax.reshape(b, s, 1),
        logits.reshape(b, s, e),
    )


# speedup vs baseline: 1.0474x; 1.0259x over previous
import jax
import jax.numpy as jnp
from jax.experimental import pallas as pl

_BS = 1024


def _router_kernel(x_ref, w_ref, logits_ref, onehot_ref, pmax_ref):
    xb = x_ref[0].astype(jnp.bfloat16)
    wb = w_ref[...].astype(jnp.bfloat16)
    logits = jax.lax.dot_general(
        xb, wb, (((1,), (1,)), ((), ())),
        preferred_element_type=jnp.float32)
    rmax = jnp.max(logits, axis=1, keepdims=True)
    ssum = jnp.sum(jnp.exp(logits - rmax), axis=1, keepdims=True)
    pmax_ref[0] = 1.0 / ssum
    e = logits.shape[1]
    iota = jax.lax.broadcasted_iota(jnp.int32, logits.shape, 1)
    idx = jnp.min(jnp.where(logits == rmax, iota, e), axis=1, keepdims=True)
    onehot_ref[0] = (iota == idx).astype(jnp.int32)
    logits_ref[0] = logits


def kernel(hidden_states, W):
    b, s, h = hidden_states.shape
    e = W.shape[0]

    logits, onehot, pmax = pl.pallas_call(
        _router_kernel,
        grid=(b, s // _BS),
        in_specs=[
            pl.BlockSpec((1, _BS, h), lambda i, j: (i, j, 0)),
            pl.BlockSpec((e, h), lambda i, j: (0, 0)),
        ],
        out_specs=[
            pl.BlockSpec((1, _BS, e), lambda i, j: (i, j, 0)),
            pl.BlockSpec((1, _BS, e), lambda i, j: (i, j, 0)),
            pl.BlockSpec((1, _BS, 1), lambda i, j: (i, j, 0)),
        ],
        out_shape=[
            jax.ShapeDtypeStruct((b, s, e), jnp.float32),
            jax.ShapeDtypeStruct((b, s, e), jnp.int32),
            jax.ShapeDtypeStruct((b, s, 1), jnp.float32),
        ],
    )(hidden_states, W)

    return (onehot, pmax, logits)
